# baseline (speedup 1.0000x reference)
import functools

import jax
import jax.numpy as jnp
from jax import lax
from jax.experimental import pallas as pl
from jax.experimental.pallas import tpu as pltpu
from jax.experimental.pallas import tpu_sc as plsc

B = 16384
V = 100000
NUM_CORES = 2
NUM_SUBCORES = 16
NW = NUM_CORES * NUM_SUBCORES
BPW = B // NW
RBLK = 4096
BLK = 2048

_F32 = jnp.float32
_HI = lax.Precision.HIGHEST


def _repack_body(umf_t, imf_t, umlp_t, imlp_t, out_r):
    stacked = jnp.concatenate(
        [umf_t[...], imf_t[...], umlp_t[...], imlp_t[...]], axis=0)
    out_r[:, 0:96] = jnp.swapaxes(stacked, 0, 1)


def _repack(umf_t, imf_t, umlp_t, imlp_t):
    grid = (pl.cdiv(V, RBLK),)
    spec = lambda d: pl.BlockSpec((d, RBLK), lambda b: (0, b))
    return pl.pallas_call(
        _repack_body,
        grid=grid,
        in_specs=[spec(16), spec(16), spec(32), spec(32)],
        out_specs=pl.BlockSpec((RBLK, 128), lambda b: (b, 0)),
        out_shape=jax.ShapeDtypeStruct((V, 128), _F32),
        compiler_params=pltpu.CompilerParams(
            dimension_semantics=("arbitrary",)),
    )(umf_t, imf_t, umlp_t, imlp_t)


def _gather(combined, u, i):
    mesh = plsc.VectorSubcoreMesh(core_axis_name="c", subcore_axis_name="s")

    @functools.partial(
        pl.kernel,
        mesh=mesh,
        compiler_params=pltpu.CompilerParams(use_tc_tiling_on_sc=True),
        out_type=(
            jax.ShapeDtypeStruct((B, 128), _F32),
            jax.ShapeDtypeStruct((B, 128), _F32),
        ),
        scratch_types=[
            pltpu.VMEM((BPW,), jnp.int32),
            pltpu.VMEM((BPW,), jnp.int32),
            pltpu.VMEM((BPW // 2, 128), _F32),
            pltpu.VMEM((BPW // 2, 128), _F32),
            pltpu.SemaphoreType.DMA,
            pltpu.SemaphoreType.DMA,
        ],
    )
    def k(c_hbm, u_hbm, i_hbm, o_u, o_i, uidx, iidx, buf_a, buf_b, sem_a,
          sem_b):
        wid = lax.axis_index("s") * NUM_CORES + lax.axis_index("c")
        base = wid * BPW
        half = BPW // 2
        pltpu.sync_copy(u_hbm.at[pl.ds(base, BPW)], uidx)
        pltpu.sync_copy(i_hbm.at[pl.ds(base, BPW)], iidx)
        c0 = pltpu.async_copy(c_hbm.at[uidx.at[pl.ds(0, half)]], buf_a, sem_a)
        c1 = pltpu.async_copy(c_hbm.at[uidx.at[pl.ds(half, half)]], buf_b,
                              sem_b)
        c0.wait()
        pltpu.sync_copy(buf_a, o_u.at[pl.ds(base, half)])
        c2 = pltpu.async_copy(c_hbm.at[iidx.at[pl.ds(0, half)]], buf_a, sem_a)
        c1.wait()
        pltpu.sync_copy(buf_b, o_u.at[pl.ds(base + half, half)])
        c3 = pltpu.async_copy(c_hbm.at[iidx.at[pl.ds(half, half)]], buf_b,
                              sem_b)
        c2.wait()
        pltpu.sync_copy(buf_a, o_i.at[pl.ds(base, half)])
        c3.wait()
        pltpu.sync_copy(buf_b, o_i.at[pl.ds(base + half, half)])

    return k(combined, u, i)


def _dense_body(urows, irows, text_t, wt_t, bt_r, w1_t, b1_r, w2_t, b2_r,
                wp_t, bp_r, out_r):
    bf = jnp.bfloat16
    f32 = jnp.float32
    def dot_t(a, w_t):
        return lax.dot_general(a.astype(bf), w_t.astype(bf),
                               (((1,), (1,)), ((), ())),
                               preferred_element_type=f32)
    x1 = lax.dot_general(text_t[...].astype(bf), wt_t[...].astype(bf),
                         (((0,), (1,)), ((), ())), preferred_element_type=f32)
    tv = jnp.maximum(x1 + bt_r[...], 0.0)
    umlp = urows[:, 32:64]
    imlp = irows[:, 64:96]
    h = (dot_t(umlp, w1_t[:, 0:32]) + dot_t(imlp, w1_t[:, 32:64])
         + dot_t(tv, w1_t[:, 64:80]) + b1_r[...])
    h = jnp.maximum(h, 0.0)
    h = jnp.maximum(dot_t(h, w2_t[...]) + b2_r[...], 0.0)
    mf = urows[:, 0:16] * irows[:, 16:32]
    logit = jnp.sum(mf * wp_t[:, 0:16] + h * wp_t[:, 16:32], axis=1) + bp_r[0, 0]
    out_r[...] = jax.nn.sigmoid(logit)


def _dense(urows, irows, text_t, wt_t, bt, w1_t, b1, w2_t, b2, wp_t, bp):
    grid = (B // BLK,)
    full = lambda shape: pl.BlockSpec(shape, lambda b: (0, 0))
    return pl.pallas_call(
        _dense_body,
        grid=grid,
        in_specs=[pl.BlockSpec((BLK, 128), lambda b: (b, 0)),
                  pl.BlockSpec((BLK, 128), lambda b: (b, 0)),
                  pl.BlockSpec((50, BLK), lambda b: (0, b)),
                  full((16, 50)), full((1, 16)), full((32, 80)),
                  full((1, 32)), full((16, 32)), full((1, 16)),
                  full((1, 32)), full((1, 1))],
        out_specs=pl.BlockSpec((BLK,), lambda b: (b,)),
        out_shape=jax.ShapeDtypeStruct((B,), _F32),
        compiler_params=pltpu.CompilerParams(
            dimension_semantics=("arbitrary",),
            fuse_transposed_lhs_in_matmul=True),
    )(urows, irows, text_t, wt_t, bt, w1_t, b1, w2_t, b2, wp_t, bp)


def kernel(u, i, text_features, emb_user_mf, emb_item_mf, emb_user_mlp,
           emb_item_mlp, W_text, b_text, W1, b1, W2, b2, Wp, bp):
    combined = _repack(emb_user_mf.T, emb_item_mf.T,
                       emb_user_mlp.T, emb_item_mlp.T)
    urows, irows = _gather(combined, u, i)
    return _dense(urows, irows, text_features.T,
                  W_text.T, b_text.reshape(1, 16), W1.T, b1.reshape(1, 32),
                  W2.T, b2.reshape(1, 16), Wp.T, bp.reshape(1, 1))


# speedup vs baseline: 1.0087x; 1.0087x over previous
import functools

import jax
import jax.numpy as jnp
from jax import lax
from jax.experimental import pallas as pl
from jax.experimental.pallas import tpu as pltpu
from jax.experimental.pallas import tpu_sc as plsc

B = 16384
V = 100000
NUM_CORES = 2
NUM_SUBCORES = 16
NW = NUM_CORES * NUM_SUBCORES
BPW = B // NW
RBLK = 4096
BLK = 2048
TBLK = 8192

_F32 = jnp.float32
_HI = lax.Precision.HIGHEST


def _repack_body(umf_t, imf_t, umlp_t, imlp_t, out_r):
    stacked = jnp.concatenate(
        [umf_t[...], imf_t[...], umlp_t[...], imlp_t[...]], axis=0)
    out_r[:, 0:96] = jnp.swapaxes(stacked, 0, 1)


def _repack(umf_t, imf_t, umlp_t, imlp_t):
    grid = (pl.cdiv(V, RBLK),)
    spec = lambda d: pl.BlockSpec((d, RBLK), lambda b: (0, b))
    return pl.pallas_call(
        _repack_body,
        grid=grid,
        in_specs=[spec(16), spec(16), spec(32), spec(32)],
        out_specs=pl.BlockSpec((RBLK, 128), lambda b: (b, 0)),
        out_shape=jax.ShapeDtypeStruct((V, 128), _F32),
        compiler_params=pltpu.CompilerParams(
            dimension_semantics=("arbitrary",)),
    )(umf_t, imf_t, umlp_t, imlp_t)


def _gather(combined, u, i):
    mesh = plsc.VectorSubcoreMesh(core_axis_name="c", subcore_axis_name="s")

    @functools.partial(
        pl.kernel,
        mesh=mesh,
        compiler_params=pltpu.CompilerParams(use_tc_tiling_on_sc=True),
        out_type=(
            jax.ShapeDtypeStruct((B, 128), _F32),
            jax.ShapeDtypeStruct((B, 128), _F32),
        ),
        scratch_types=[
            pltpu.VMEM((BPW,), jnp.int32),
            pltpu.VMEM((BPW,), jnp.int32),
            pltpu.VMEM((BPW // 2, 128), _F32),
            pltpu.VMEM((BPW // 2, 128), _F32),
            pltpu.SemaphoreType.DMA,
            pltpu.SemaphoreType.DMA,
        ],
    )
    def k(c_hbm, u_hbm, i_hbm, o_u, o_i, uidx, iidx, buf_a, buf_b, sem_a,
          sem_b):
        wid = lax.axis_index("s") * NUM_CORES + lax.axis_index("c")
        base = wid * BPW
        half = BPW // 2
        pltpu.sync_copy(u_hbm.at[pl.ds(base, BPW)], uidx)
        pltpu.sync_copy(i_hbm.at[pl.ds(base, BPW)], iidx)
        c0 = pltpu.async_copy(c_hbm.at[uidx.at[pl.ds(0, half)]], buf_a, sem_a)
        c1 = pltpu.async_copy(c_hbm.at[uidx.at[pl.ds(half, half)]], buf_b,
                              sem_b)
        c0.wait()
        pltpu.sync_copy(buf_a, o_u.at[pl.ds(base, half)])
        c2 = pltpu.async_copy(c_hbm.at[iidx.at[pl.ds(0, half)]], buf_a, sem_a)
        c1.wait()
        pltpu.sync_copy(buf_b, o_u.at[pl.ds(base + half, half)])
        c3 = pltpu.async_copy(c_hbm.at[iidx.at[pl.ds(half, half)]], buf_b,
                              sem_b)
        c2.wait()
        pltpu.sync_copy(buf_a, o_i.at[pl.ds(base, half)])
        c3.wait()
        pltpu.sync_copy(buf_b, o_i.at[pl.ds(base + half, half)])

    return k(combined, u, i)


def _text_body(text_t, wt_t, bt_r, out_r):
    bf = jnp.bfloat16
    x1 = lax.dot_general(wt_t[...].astype(bf), text_t[...].astype(bf),
                         (((1,), (0,)), ((), ())),
                         preferred_element_type=jnp.float32)
    bt_col = jnp.swapaxes(bt_r[...], 0, 1)
    out_r[...] = jnp.maximum(x1 + bt_col, 0.0)


def _text(text_t, wt_t, bt):
    grid = (B // TBLK,)
    return pl.pallas_call(
        _text_body,
        grid=grid,
        in_specs=[pl.BlockSpec((50, TBLK), lambda b: (0, b)),
                  pl.BlockSpec((16, 50), lambda b: (0, 0)),
                  pl.BlockSpec((1, 16), lambda b: (0, 0))],
        out_specs=pl.BlockSpec((16, TBLK), lambda b: (0, b)),
        out_shape=jax.ShapeDtypeStruct((16, B), _F32),
        compiler_params=pltpu.CompilerParams(
            dimension_semantics=("arbitrary",)),
    )(text_t, wt_t, bt)


def _dense_body(urows, irows, tvt, w1_t, b1_r, w2_t, b2_r, wp_t, bp_r, out_r):
    bf = jnp.bfloat16
    f32 = jnp.float32
    def dot_t(a, w_t):
        return lax.dot_general(a.astype(bf), w_t.astype(bf),
                               (((1,), (1,)), ((), ())),
                               preferred_element_type=f32)
    umlp = urows[:, 32:64]
    imlp = irows[:, 64:96]
    h = (dot_t(umlp, w1_t[:, 0:32]) + dot_t(imlp, w1_t[:, 32:64])
         + lax.dot_general(tvt[...].astype(bf), w1_t[:, 64:80].astype(bf),
                           (((0,), (1,)), ((), ())),
                           preferred_element_type=f32)
         + b1_r[...])
    h = jnp.maximum(h, 0.0)
    h = jnp.maximum(dot_t(h, w2_t[...]) + b2_r[...], 0.0)
    mf = urows[:, 0:16] * irows[:, 16:32]
    logit = jnp.sum(mf * wp_t[:, 0:16] + h * wp_t[:, 16:32], axis=1) + bp_r[0, 0]
    out_r[...] = jax.nn.sigmoid(logit)


def _dense(urows, irows, tvt, w1_t, b1, w2_t, b2, wp_t, bp):
    grid = (B // BLK,)
    full = lambda shape: pl.BlockSpec(shape, lambda b: (0, 0))
    return pl.pallas_call(
        _dense_body,
        grid=grid,
        in_specs=[pl.BlockSpec((BLK, 128), lambda b: (b, 0)),
                  pl.BlockSpec((BLK, 128), lambda b: (b, 0)),
                  pl.BlockSpec((16, BLK), lambda b: (0, b)),
                  full((32, 80)), full((1, 32)), full((16, 32)),
                  full((1, 16)), full((1, 32)), full((1, 1))],
        out_specs=pl.BlockSpec((BLK,), lambda b: (b,)),
        out_shape=jax.ShapeDtypeStruct((B,), _F32),
        compiler_params=pltpu.CompilerParams(
            dimension_semantics=("arbitrary",),
            fuse_transposed_lhs_in_matmul=True),
    )(urows, irows, tvt, w1_t, b1, w2_t, b2, wp_t, bp)


def kernel(u, i, text_features, emb_user_mf, emb_item_mf, emb_user_mlp,
           emb_item_mlp, W_text, b_text, W1, b1, W2, b2, Wp, bp):
    combined = _repack(emb_user_mf.T, emb_item_mf.T,
                       emb_user_mlp.T, emb_item_mlp.T)
    urows, irows = _gather(combined, u, i)
    tvt = _text(text_features.T, W_text.T, b_text.reshape(1, 16))
    return _dense(urows, irows, tvt,
                  W1.T, b1.reshape(1, 32),
                  W2.T, b2.reshape(1, 16), Wp.T, bp.reshape(1, 1))


# speedup vs baseline: 1.0836x; 1.0743x over previous
import functools

import jax
import jax.numpy as jnp
from jax import lax
from jax.experimental import pallas as pl
from jax.experimental.pallas import tpu as pltpu
from jax.experimental.pallas import tpu_sc as plsc

B = 16384
V = 100000
NUM_CORES = 2
NUM_SUBCORES = 16
NW = NUM_CORES * NUM_SUBCORES
BPW = B // NW
RBLK = 4096
BLK = 2048
TBLK = 8192

_F32 = jnp.float32
_HI = lax.Precision.HIGHEST


def _repack_body(umf_t, imf_t, umlp_t, imlp_t, out_r):
    stacked = jnp.concatenate(
        [umf_t[...], imf_t[...], umlp_t[...], imlp_t[...]], axis=0)
    out_r[:, 0:96] = jnp.swapaxes(stacked, 0, 1)


def _repack(umf_t, imf_t, umlp_t, imlp_t):
    grid = (pl.cdiv(V, RBLK),)
    spec = lambda d: pl.BlockSpec((d, RBLK), lambda b: (0, b))
    return pl.pallas_call(
        _repack_body,
        grid=grid,
        in_specs=[spec(16), spec(16), spec(32), spec(32)],
        out_specs=pl.BlockSpec((RBLK, 128), lambda b: (b, 0)),
        out_shape=jax.ShapeDtypeStruct((V, 128), _F32),
        compiler_params=pltpu.CompilerParams(
            dimension_semantics=("arbitrary",)),
    )(umf_t, imf_t, umlp_t, imlp_t)


def _gather(combined, u, i):
    mesh = plsc.VectorSubcoreMesh(core_axis_name="c", subcore_axis_name="s")

    @functools.partial(
        pl.kernel,
        mesh=mesh,
        compiler_params=pltpu.CompilerParams(use_tc_tiling_on_sc=True),
        out_type=(
            jax.ShapeDtypeStruct((B, 128), _F32),
            jax.ShapeDtypeStruct((B, 128), _F32),
        ),
        scratch_types=[
            pltpu.VMEM((BPW,), jnp.int32),
            pltpu.VMEM((BPW,), jnp.int32),
            pltpu.VMEM((BPW // 2, 128), _F32),
            pltpu.VMEM((BPW // 2, 128), _F32),
            pltpu.SemaphoreType.DMA,
            pltpu.SemaphoreType.DMA,
        ],
    )
    def k(c_hbm, u_hbm, i_hbm, o_u, o_i, uidx, iidx, buf_a, buf_b, sem_a,
          sem_b):
        wid = lax.axis_index("s") * NUM_CORES + lax.axis_index("c")
        base = wid * BPW
        half = BPW // 2
        pltpu.sync_copy(u_hbm.at[pl.ds(base, BPW)], uidx)
        pltpu.sync_copy(i_hbm.at[pl.ds(base, BPW)], iidx)
        c0 = pltpu.async_copy(c_hbm.at[uidx.at[pl.ds(0, half)]], buf_a, sem_a)
        c1 = pltpu.async_copy(c_hbm.at[uidx.at[pl.ds(half, half)]], buf_b,
                              sem_b)
        c0.wait()
        pltpu.sync_copy(buf_a, o_u.at[pl.ds(base, half)])
        c2 = pltpu.async_copy(c_hbm.at[iidx.at[pl.ds(0, half)]], buf_a, sem_a)
        c1.wait()
        pltpu.sync_copy(buf_b, o_u.at[pl.ds(base + half, half)])
        c3 = pltpu.async_copy(c_hbm.at[iidx.at[pl.ds(half, half)]], buf_b,
                              sem_b)
        c2.wait()
        pltpu.sync_copy(buf_a, o_i.at[pl.ds(base, half)])
        c3.wait()
        pltpu.sync_copy(buf_b, o_i.at[pl.ds(base + half, half)])

    return k(combined, u, i)


def _text_body(text_t, wt_t, bt_r, out_r):
    bf = jnp.bfloat16
    x1 = lax.dot_general(wt_t[...].astype(bf), text_t[...].astype(bf),
                         (((1,), (0,)), ((), ())),
                         preferred_element_type=jnp.float32)
    bt_col = jnp.swapaxes(bt_r[...], 0, 1)
    out_r[...] = jnp.maximum(x1 + bt_col, 0.0)


def _text(text_t, wt_t, bt):
    grid = (B // TBLK,)
    return pl.pallas_call(
        _text_body,
        grid=grid,
        in_specs=[pl.BlockSpec((50, TBLK), lambda b: (0, b)),
                  pl.BlockSpec((16, 50), lambda b: (0, 0)),
                  pl.BlockSpec((1, 16), lambda b: (0, 0))],
        out_specs=pl.BlockSpec((16, TBLK), lambda b: (0, b)),
        out_shape=jax.ShapeDtypeStruct((16, B), _F32),
        compiler_params=pltpu.CompilerParams(
            dimension_semantics=("arbitrary",)),
    )(text_t, wt_t, bt)


def _dense_body(urows, irows, tvt, w1_t, b1_r, w2_t, b2_r, wp_t, bp_r, out_r):
    bf = jnp.bfloat16
    f32 = jnp.float32
    def dot_t(a, w_t):
        return lax.dot_general(a.astype(bf), w_t.astype(bf),
                               (((1,), (1,)), ((), ())),
                               preferred_element_type=f32)
    umlp = urows[:, 32:64]
    imlp = irows[:, 64:96]
    h = (dot_t(umlp, w1_t[:, 0:32]) + dot_t(imlp, w1_t[:, 32:64])
         + lax.dot_general(tvt[...].astype(bf), w1_t[:, 64:80].astype(bf),
                           (((0,), (1,)), ((), ())),
                           preferred_element_type=f32)
         + b1_r[...])
    h = jnp.maximum(h, 0.0)
    h = jnp.maximum(dot_t(h, w2_t[...]) + b2_r[...], 0.0)
    mf = urows[:, 0:16] * irows[:, 16:32]
    logit = jnp.sum(mf * wp_t[:, 0:16] + h * wp_t[:, 16:32], axis=1) + bp_r[0, 0]
    out_r[...] = jax.nn.sigmoid(logit).reshape(BLK // 128, 128)


def _dense(urows, irows, tvt, w1_t, b1, w2_t, b2, wp_t, bp):
    grid = (B // BLK,)
    full = lambda shape: pl.BlockSpec(shape, lambda b: (0, 0))
    return pl.pallas_call(
        _dense_body,
        grid=grid,
        in_specs=[pl.BlockSpec((BLK, 128), lambda b: (b, 0)),
                  pl.BlockSpec((BLK, 128), lambda b: (b, 0)),
                  pl.BlockSpec((16, BLK), lambda b: (0, b)),
                  full((32, 80)), full((1, 32)), full((16, 32)),
                  full((1, 16)), full((1, 32)), full((1, 1))],
        out_specs=pl.BlockSpec((BLK // 128, 128), lambda b: (b, 0)),
        out_shape=jax.ShapeDtypeStruct((B // 128, 128), _F32),
        compiler_params=pltpu.CompilerParams(
            dimension_semantics=("arbitrary",),
            fuse_transposed_lhs_in_matmul=True),
    )(urows, irows, tvt, w1_t, b1, w2_t, b2, wp_t, bp).reshape(B)


def kernel(u, i, text_features, emb_user_mf, emb_item_mf, emb_user_mlp,
           emb_item_mlp, W_text, b_text, W1, b1, W2, b2, Wp, bp):
    combined = _repack(emb_user_mf.T, emb_item_mf.T,
                       emb_user_mlp.T, emb_item_mlp.T)
    urows, irows = _gather(combined, u, i)
    tvt = _text(text_features.T, W_text.T, b_text.reshape(1, 16))
    return _dense(urows, irows, tvt,
                  W1.T, b1.reshape(1, 32),
                  W2.T, b2.reshape(1, 16), Wp.T, bp.reshape(1, 1))


# speedup vs baseline: 1.1748x; 1.0841x over previous
import functools

import jax
import jax.numpy as jnp
from jax import lax
from jax.experimental import pallas as pl
from jax.experimental.pallas import tpu as pltpu
from jax.experimental.pallas import tpu_sc as plsc

B = 16384
V = 100000
NUM_CORES = 2
NUM_SUBCORES = 16
NW = NUM_CORES * NUM_SUBCORES
BPW = B // NW
RBLK = 8192
BLK = 2048
TBLK = 8192

_F32 = jnp.float32
_HI = lax.Precision.HIGHEST


def _repack_body(umf_t, imf_t, umlp_t, imlp_t, out_r):
    stacked = jnp.concatenate(
        [umf_t[...], imf_t[...], umlp_t[...], imlp_t[...]], axis=0)
    out_r[:, 0:96] = jnp.swapaxes(stacked, 0, 1)


def _repack(umf_t, imf_t, umlp_t, imlp_t):
    grid = (pl.cdiv(V, RBLK),)
    spec = lambda d: pl.BlockSpec((d, RBLK), lambda b: (0, b))
    return pl.pallas_call(
        _repack_body,
        grid=grid,
        in_specs=[spec(16), spec(16), spec(32), spec(32)],
        out_specs=pl.BlockSpec((RBLK, 128), lambda b: (b, 0)),
        out_shape=jax.ShapeDtypeStruct((V, 128), _F32),
        compiler_params=pltpu.CompilerParams(
            dimension_semantics=("arbitrary",)),
    )(umf_t, imf_t, umlp_t, imlp_t)


def _gather(combined, u, i):
    mesh = plsc.VectorSubcoreMesh(core_axis_name="c", subcore_axis_name="s")

    @functools.partial(
        pl.kernel,
        mesh=mesh,
        compiler_params=pltpu.CompilerParams(use_tc_tiling_on_sc=True),
        out_type=(
            jax.ShapeDtypeStruct((B, 128), _F32),
            jax.ShapeDtypeStruct((B, 128), _F32),
        ),
        scratch_types=[
            pltpu.VMEM((BPW,), jnp.int32),
            pltpu.VMEM((BPW,), jnp.int32),
            pltpu.VMEM((BPW // 2, 128), _F32),
            pltpu.VMEM((BPW // 2, 128), _F32),
            pltpu.SemaphoreType.DMA,
            pltpu.SemaphoreType.DMA,
        ],
    )
    def k(c_hbm, u_hbm, i_hbm, o_u, o_i, uidx, iidx, buf_a, buf_b, sem_a,
          sem_b):
        wid = lax.axis_index("s") * NUM_CORES + lax.axis_index("c")
        base = wid * BPW
        half = BPW // 2
        pltpu.sync_copy(u_hbm.at[pl.ds(base, BPW)], uidx)
        pltpu.sync_copy(i_hbm.at[pl.ds(base, BPW)], iidx)
        c0 = pltpu.async_copy(c_hbm.at[uidx.at[pl.ds(0, half)]], buf_a, sem_a)
        c1 = pltpu.async_copy(c_hbm.at[uidx.at[pl.ds(half, half)]], buf_b,
                              sem_b)
        c0.wait()
        pltpu.sync_copy(buf_a, o_u.at[pl.ds(base, half)])
        c2 = pltpu.async_copy(c_hbm.at[iidx.at[pl.ds(0, half)]], buf_a, sem_a)
        c1.wait()
        pltpu.sync_copy(buf_b, o_u.at[pl.ds(base + half, half)])
        c3 = pltpu.async_copy(c_hbm.at[iidx.at[pl.ds(half, half)]], buf_b,
                              sem_b)
        c2.wait()
        pltpu.sync_copy(buf_a, o_i.at[pl.ds(base, half)])
        c3.wait()
        pltpu.sync_copy(buf_b, o_i.at[pl.ds(base + half, half)])

    return k(combined, u, i)


def _text_body(text_t, wt_t, bt_r, out_r):
    bf = jnp.bfloat16
    x1 = lax.dot_general(wt_t[...].astype(bf), text_t[...].astype(bf),
                         (((1,), (0,)), ((), ())),
                         preferred_element_type=jnp.float32)
    bt_col = jnp.swapaxes(bt_r[...], 0, 1)
    out_r[...] = jnp.maximum(x1 + bt_col, 0.0)


def _text(text_t, wt_t, bt):
    grid = (B // TBLK,)
    return pl.pallas_call(
        _text_body,
        grid=grid,
        in_specs=[pl.BlockSpec((50, TBLK), lambda b: (0, b)),
                  pl.BlockSpec((16, 50), lambda b: (0, 0)),
                  pl.BlockSpec((1, 16), lambda b: (0, 0))],
        out_specs=pl.BlockSpec((16, TBLK), lambda b: (0, b)),
        out_shape=jax.ShapeDtypeStruct((16, B), _F32),
        compiler_params=pltpu.CompilerParams(
            dimension_semantics=("arbitrary",)),
    )(text_t, wt_t, bt)


def _dense_body(urows, irows, tvt, w1_t, b1_r, w2_t, b2_r, wp_t, bp_r, out_r):
    bf = jnp.bfloat16
    f32 = jnp.float32
    def dot_t(a, w_t):
        return lax.dot_general(a.astype(bf), w_t.astype(bf),
                               (((1,), (1,)), ((), ())),
                               preferred_element_type=f32)
    umlp = urows[:, 32:64]
    imlp = irows[:, 64:96]
    h = (dot_t(umlp, w1_t[:, 0:32]) + dot_t(imlp, w1_t[:, 32:64])
         + lax.dot_general(tvt[...].astype(bf), w1_t[:, 64:80].astype(bf),
                           (((0,), (1,)), ((), ())),
                           preferred_element_type=f32)
         + b1_r[...])
    h = jnp.maximum(h, 0.0)
    h = jnp.maximum(dot_t(h, w2_t[...]) + b2_r[...], 0.0)
    mf = urows[:, 0:16] * irows[:, 16:32]
    logit = jnp.sum(mf * wp_t[:, 0:16] + h * wp_t[:, 16:32], axis=1) + bp_r[0, 0]
    out_r[...] = jax.nn.sigmoid(logit).reshape(BLK // 128, 128)


def _dense(urows, irows, tvt, w1_t, b1, w2_t, b2, wp_t, bp):
    grid = (B // BLK,)
    full = lambda shape: pl.BlockSpec(shape, lambda b: (0, 0))
    return pl.pallas_call(
        _dense_body,
        grid=grid,
        in_specs=[pl.BlockSpec((BLK, 128), lambda b: (b, 0)),
                  pl.BlockSpec((BLK, 128), lambda b: (b, 0)),
                  pl.BlockSpec((16, BLK), lambda b: (0, b)),
                  full((32, 80)), full((1, 32)), full((16, 32)),
                  full((1, 16)), full((1, 32)), full((1, 1))],
        out_specs=pl.BlockSpec((BLK // 128, 128), lambda b: (b, 0)),
        out_shape=jax.ShapeDtypeStruct((B // 128, 128), _F32),
        compiler_params=pltpu.CompilerParams(
            dimension_semantics=("arbitrary",),
            fuse_transposed_lhs_in_matmul=True),
    )(urows, irows, tvt, w1_t, b1, w2_t, b2, wp_t, bp).reshape(B)


def kernel(u, i, text_features, emb_user_mf, emb_item_mf, emb_user_mlp,
           emb_item_mlp, W_text, b_text, W1, b1, W2, b2, Wp, bp):
    combined = _repack(emb_user_mf.T, emb_item_mf.T,
                       emb_user_mlp.T, emb_item_mlp.T)
    urows, irows = _gather(combined, u, i)
    tvt = _text(text_features.T, W_text.T, b_text.reshape(1, 16))
    return _dense(urows, irows, tvt,
                  W1.T, b1.reshape(1, 32),
                  W2.T, b2.reshape(1, 16), Wp.T, bp.reshape(1, 1))
